# baseline (speedup 1.0000x reference)
import functools

import jax
import jax.numpy as jnp
from jax import lax
from jax.experimental import pallas as pl
from jax.experimental.pallas import tpu as pltpu
from jax.experimental.pallas import tpu_sc as plsc

_VOCAB = 1000000
_D = 32
_B = 16384
_H = 50
_NW = 32
_ROWS_W = _B // _NW
_K = 8
_NJ = _ROWS_W // _K

_mesh = plsc.VectorSubcoreMesh(core_axis_name="c", subcore_axis_name="s")


@functools.partial(
    pl.kernel,
    mesh=_mesh,
    out_type=jax.ShapeDtypeStruct((_B, _H, _D), jnp.float32),
    scratch_types=(
        [pltpu.VMEM((_ROWS_W, _H), jnp.int32)]
        + [pltpu.VMEM((_H, _D), jnp.float32) for _ in range(_K)]
        + [pltpu.SemaphoreType.DMA for _ in range(2 * _K)]
    ),
    compiler_params=pltpu.CompilerParams(use_tc_tiling_on_sc=False),
)
def _emb_sum(data_hbm, w_hbm, p_hbm, s_hbm, out_hbm, idx_v, *scratch):
    bufs = scratch[:_K]
    gsems = scratch[_K:2 * _K]
    osems = scratch[2 * _K:]

    wid = lax.axis_index("s") * 2 + lax.axis_index("c")
    base = wid * _ROWS_W
    pltpu.sync_copy(data_hbm.at[pl.ds(base, _ROWS_W)], idx_v)

    def fire_w(k, r):
        return pltpu.async_copy(w_hbm.at[idx_v.at[r]], bufs[k], gsems[k])

    def fire_p(k, r):
        return pltpu.async_copy(
            p_hbm.at[idx_v.at[r]], bufs[k], gsems[k], add=True)

    def fire_s(k, r):
        return pltpu.async_copy(
            s_hbm.at[idx_v.at[r]], bufs[k], gsems[k], add=True)

    def fire_out(k, r):
        return pltpu.async_copy(bufs[k], out_hbm.at[base + r], osems[k])

    def drain(k, n):
        for _ in range(n):
            pltpu.make_async_copy(
                w_hbm.at[pl.ds(0, _H)], bufs[k], gsems[k]).wait()

    descs = [fire_w(k, k) for k in range(_K)]
    pds = []
    for k in range(_K):
        descs[k].wait()
        pds.append(fire_p(k, k))
    for k in range(_K):
        pds[k].wait()
        fire_s(k, k)

    def body(j, _):
        outs = []
        for k in range(_K):
            drain(k, 1)
            outs.append(fire_out(k, (j - 1) * _K + k))
        wds = []
        for k in range(_K):
            outs[k].wait()
            wds.append(fire_w(k, j * _K + k))
        pds = []
        for k in range(_K):
            wds[k].wait()
            pds.append(fire_p(k, j * _K + k))
        for k in range(_K):
            pds[k].wait()
            fire_s(k, j * _K + k)
        return ()

    lax.fori_loop(1, _NJ, body, ())

    outs = []
    for k in range(_K):
        drain(k, 1)
        outs.append(fire_out(k, (_NJ - 1) * _K + k))
    for k in range(_K):
        outs[k].wait()


def kernel(data, W_word, W_pre, W_suf):
    return _emb_sum(data, W_word, W_pre, W_suf)
